# baseline (speedup 1.0000x reference)
import functools

import jax
import jax.numpy as jnp
from jax import lax
from jax.experimental import pallas as pl
from jax.experimental.pallas import tpu as pltpu
from jax.experimental.pallas import tpu_sc as plsc

EPS = 1e-8
COORDS_WEIGHT = 1.0

NC = 2
NS = 16
NW = NC * NS


def _pack_bf16(mm):
    lo = lax.bitcast_convert_type(mm[:, 0:64].astype(jnp.bfloat16),
                                  jnp.uint16).astype(jnp.uint32)
    hi = lax.bitcast_convert_type(mm[:, 64:128].astype(jnp.bfloat16),
                                  jnp.uint16).astype(jnp.uint32)
    return lax.bitcast_convert_type(lo | (hi << 16), jnp.int32)


def _k0_body(h_ref, cpad_ref, w1a_ref, w1b_ref, ta_ref, tb_ref):
    h = h_ref[...]
    cbits = lax.bitcast_convert_type(cpad_ref[...], jnp.int32)
    mma = jnp.dot(h, w1a_ref[...], preferred_element_type=jnp.float32)
    mmb = jnp.dot(h, w1b_ref[...], preferred_element_type=jnp.float32)
    ta_ref[...] = jnp.concatenate([_pack_bf16(mma), cbits], axis=1)
    tb_ref[...] = jnp.concatenate([_pack_bf16(mmb), cbits], axis=1)


def _prep_tables(h, cpad, w1a, w1b, blk):
    n, d = h.shape
    return pl.pallas_call(
        _k0_body,
        grid=(n // blk,),
        in_specs=[
            pl.BlockSpec((blk, d), lambda i: (i, 0)),
            pl.BlockSpec((blk, 16), lambda i: (i, 0)),
            pl.BlockSpec((d, d), lambda i: (0, 0)),
            pl.BlockSpec((d, d), lambda i: (0, 0)),
        ],
        out_specs=[
            pl.BlockSpec((blk, 80), lambda i: (i, 0)),
            pl.BlockSpec((blk, 80), lambda i: (i, 0)),
        ],
        out_shape=[
            jax.ShapeDtypeStruct((n, 80), jnp.int32),
            jax.ShapeDtypeStruct((n, 80), jnp.int32),
        ],
    )(h, cpad, w1a, w1b)


def _gather_sum(ta, tb, row_p, col_p, ep, be, frac0_num=1, frac0_den=2):
    e0 = (ep * frac0_num // frac0_den) // (NS * 2 * be) * (NS * 2 * be)
    ew0 = e0 // NS
    ew1 = (ep - e0) // NS
    nb0 = ew0 // be
    nb1 = ew1 // be
    mesh = plsc.VectorSubcoreMesh(core_axis_name="c", subcore_axis_name="s")

    @functools.partial(
        pl.kernel,
        out_type=[
            jax.ShapeDtypeStruct((ep, 80), jnp.int32),
            jax.ShapeDtypeStruct((ep, 80), jnp.int32),
        ],
        mesh=mesh,
        scratch_types=[
            pltpu.VMEM((be,), jnp.int32), pltpu.VMEM((be,), jnp.int32),
            pltpu.VMEM((be,), jnp.int32), pltpu.VMEM((be,), jnp.int32),
            pltpu.VMEM((be, 80), jnp.int32), pltpu.VMEM((be, 80), jnp.int32),
            pltpu.VMEM((be, 80), jnp.int32), pltpu.VMEM((be, 80), jnp.int32),
            pltpu.SemaphoreType.DMA, pltpu.SemaphoreType.DMA,
        ],
        compiler_params=pltpu.CompilerParams(use_tc_tiling_on_sc=False),
    )
    def k1(ta_hbm, tb_hbm, row_hbm, col_hbm, sa_hbm, sb_hbm,
           idx_r0, idx_c0, idx_r1, idx_c1, bufa0, bufb0, bufa1, bufb1,
           sem0, sem1):
        c = lax.axis_index("c")
        s = lax.axis_index("s")
        idx = ((idx_r0, idx_c0), (idx_r1, idx_c1))
        buf = ((bufa0, bufb0), (bufa1, bufb1))
        sem = (sem0, sem1)

        def load(k, off):
            pltpu.sync_copy(row_hbm.at[pl.ds(off, be)], idx[k][0])
            pltpu.sync_copy(col_hbm.at[pl.ds(off, be)], idx[k][1])
            pltpu.async_copy(ta_hbm.at[idx[k][0]], buf[k][0], sem[k])
            pltpu.async_copy(tb_hbm.at[idx[k][1]], buf[k][1], sem[k])

        def drain(k):
            pltpu.make_async_copy(ta_hbm.at[idx[k][0]], buf[k][0], sem[k]).wait()
            pltpu.make_async_copy(tb_hbm.at[idx[k][1]], buf[k][1], sem[k]).wait()

        def store(k, off):
            pltpu.sync_copy(buf[k][0], sa_hbm.at[pl.ds(off, be)])
            pltpu.sync_copy(buf[k][1], sb_hbm.at[pl.ds(off, be)])

        def run(base, nb):
            load(0, base)

            def pair(i, carry):
                off0 = base + (2 * i) * be
                load(1, off0 + be)
                drain(0)
                store(0, off0)

                @pl.when(i + 1 < nb // 2)
                def _():
                    load(0, off0 + 2 * be)

                drain(1)
                store(1, off0 + be)
                return carry

            lax.fori_loop(0, nb // 2, pair, 0)

        @pl.when(c == 0)
        def _():
            run(s * ew0, nb0)

        @pl.when(c == 1)
        def _():
            run(NS * ew0 + s * ew1, nb1)

    return k1(ta, tb, row_p, col_p)


def _unpack_bf16(p):
    lo = lax.bitcast_convert_type(p << 16, jnp.float32)
    hi = lax.bitcast_convert_type(p & jnp.int32(-65536), jnp.float32)
    return lo, hi


def _k2_body(sa_ref, sb_ref, wr01_ref, wr2_ref, be1_ref, we2_ref, be2_ref,
             wc1_ref, bc1_ref, wclt_ref, t_ref, td_ref):
    xa = sa_ref[...]
    xb = sb_ref[...]
    alo, ahi = _unpack_bf16(xa[:, 0:64])
    blo, bhi = _unpack_bf16(xb[:, 0:64])
    s = jnp.concatenate([alo + blo, ahi + bhi], axis=1)
    da = lax.bitcast_convert_type(xa[:, 64:67], jnp.float32)
    db = lax.bitcast_convert_type(xb[:, 64:67], jnp.float32)
    d = da - db
    sumsq = jnp.sum(d * d, axis=1, keepdims=True)
    nrm = jnp.sqrt(sumsq)
    fxv = sumsq / ((nrm + EPS) * (nrm + EPS))
    e1 = jax.nn.relu(s + nrm * wr01_ref[...] + fxv * wr2_ref[...] + be1_ref[...])
    ef = jax.nn.relu(
        jnp.dot(e1, we2_ref[...], preferred_element_type=jnp.float32)
        + be2_ref[...])
    c1 = jax.nn.relu(
        jnp.dot(ef, wc1_ref[...], preferred_element_type=jnp.float32)
        + bc1_ref[...])
    scalar = jnp.sum(c1 * wclt_ref[...], axis=1, keepdims=True)
    trans = jnp.clip(d * scalar, -100.0, 100.0)
    blk = s.shape[0]
    t_ref[...] = ef
    td_ref[...] = jnp.concatenate(
        [trans, jnp.ones((blk, 1), jnp.float32),
         jnp.zeros((blk, 12), jnp.float32)], axis=1)


def _edge_mlp(sa, sb, wr01, wr2, be1, we2, be2, wc1, bc1, wclt, blk):
    ep = sa.shape[0]
    wspec = pl.BlockSpec((128, 128), lambda i: (0, 0))
    bspec = pl.BlockSpec((1, 128), lambda i: (0, 0))
    espec = pl.BlockSpec((blk, 128), lambda i: (i, 0))
    pspec = pl.BlockSpec((blk, 80), lambda i: (i, 0))
    return pl.pallas_call(
        _k2_body,
        grid=(ep // blk,),
        in_specs=[pspec, pspec,
                  bspec, bspec, bspec, wspec, bspec, wspec, bspec, bspec],
        out_specs=[espec, pl.BlockSpec((blk, 16), lambda i: (i, 0))],
        out_shape=[
            jax.ShapeDtypeStruct((ep, 128), jnp.float32),
            jax.ShapeDtypeStruct((ep, 16), jnp.float32),
        ],
    )(sa, sb, wr01, wr2, be1, we2, be2, wc1, bc1, wclt)


def _segment_sum(t, td, rowscat, npad, ep, be):
    ew = ep // NW
    nb = ew // be
    rs = npad // NS
    ch = 64
    nch = rs // ch
    mesh = plsc.VectorSubcoreMesh(core_axis_name="c", subcore_axis_name="s")

    @functools.partial(
        pl.kernel,
        out_type=[
            jax.ShapeDtypeStruct((NC, npad, 128), jnp.float32),
            jax.ShapeDtypeStruct((NC, npad, 16), jnp.float32),
        ],
        mesh=mesh,
        scratch_types=[
            pltpu.VMEM((be,), jnp.int32), pltpu.VMEM((be,), jnp.int32),
            pltpu.VMEM((be, 128), jnp.float32),
            pltpu.VMEM((be, 128), jnp.float32),
            pltpu.VMEM((be, 16), jnp.float32),
            pltpu.VMEM((be, 16), jnp.float32),
            pltpu.VMEM((ch, 128), jnp.float32),
            pltpu.VMEM((ch, 16), jnp.float32),
            pltpu.VMEM_SHARED((npad, 128), jnp.float32),
            pltpu.VMEM_SHARED((npad, 16), jnp.float32),
            pltpu.SemaphoreType.DMA, pltpu.SemaphoreType.DMA,
        ],
        compiler_params=pltpu.CompilerParams(use_tc_tiling_on_sc=False),
    )
    def k3(t_hbm, td_hbm, row_hbm, outf_hbm, outd_hbm, idx_v0, idx_v1,
           buft0, buft1, bufd0, bufd1, bounce, bounced, acc, accd,
           sem0, sem1):
        c = lax.axis_index("c")
        s = lax.axis_index("s")
        base = c * (ep // NC) + s * ew
        zeros16 = jnp.zeros((16,), jnp.float32)
        idx = (idx_v0, idx_v1)
        buf = (buft0, buft1)
        bufd = (bufd0, bufd1)
        sem = (sem0, sem1)

        def zrow(i, c2):
            for j in range(8):
                bounce[i, pl.ds(j * 16, 16)] = zeros16
            bounced[i, pl.ds(0, 16)] = zeros16
            return c2

        def load(k, off):
            pltpu.async_copy(row_hbm.at[pl.ds(off, be)], idx[k], sem[k])
            pltpu.async_copy(t_hbm.at[pl.ds(off, be)], buf[k], sem[k])
            pltpu.async_copy(td_hbm.at[pl.ds(off, be)], bufd[k], sem[k])

        def drain(k, off):
            pltpu.make_async_copy(row_hbm.at[pl.ds(off, be)], idx[k],
                                  sem[k]).wait()
            pltpu.make_async_copy(t_hbm.at[pl.ds(off, be)], buf[k],
                                  sem[k]).wait()
            pltpu.make_async_copy(td_hbm.at[pl.ds(off, be)], bufd[k],
                                  sem[k]).wait()

        lax.fori_loop(0, ch, zrow, 0)

        def zch(k, c2):
            pltpu.sync_copy(bounce, acc.at[pl.ds(s * rs + k * ch, ch)])
            pltpu.sync_copy(bounced, accd.at[pl.ds(s * rs + k * ch, ch)])
            return c2

        lax.fori_loop(0, nch, zch, 0)
        plsc.subcore_barrier()
        load(0, base)

        def pair(i, c2):
            off0 = base + (2 * i) * be
            load(1, off0 + be)
            drain(0, off0)
            pltpu.sync_copy(buf[0], acc.at[idx[0]], add=True)
            pltpu.sync_copy(bufd[0], accd.at[idx[0]], add=True)

            @pl.when(i + 1 < nb // 2)
            def _():
                load(0, off0 + 2 * be)

            drain(1, off0 + be)
            pltpu.sync_copy(buf[1], acc.at[idx[1]], add=True)
            pltpu.sync_copy(bufd[1], accd.at[idx[1]], add=True)
            return c2

        lax.fori_loop(0, nb // 2, pair, 0)
        plsc.subcore_barrier()

        def och(k, c2):
            r0 = s * rs + k * ch
            pltpu.sync_copy(acc.at[pl.ds(r0, ch)], bounce)
            pltpu.sync_copy(bounce, outf_hbm.at[c, pl.ds(r0, ch)])
            pltpu.sync_copy(accd.at[pl.ds(r0, ch)], bounced)
            pltpu.sync_copy(bounced, outd_hbm.at[c, pl.ds(r0, ch)])
            return c2

        lax.fori_loop(0, nch, och, 0)

    return k3(t, td, rowscat)


def _k4_body(h_ref, pf0_ref, pf1_ref, pd0_ref, pd1_ref, wnh_ref, wna_ref,
             bn1_ref, wn2_ref, bn2_ref, hout_ref, cout_ref):
    agg = pf0_ref[...] + pf1_ref[...]
    h = h_ref[...]
    n1 = jax.nn.relu(
        jnp.dot(h, wnh_ref[...], preferred_element_type=jnp.float32)
        + jnp.dot(agg, wna_ref[...], preferred_element_type=jnp.float32)
        + bn1_ref[...])
    hout_ref[...] = (
        jnp.dot(n1, wn2_ref[...], preferred_element_type=jnp.float32)
        + bn2_ref[...])
    p = pd0_ref[...] + pd1_ref[...]
    ssum = p[:, 0:3]
    cnt = jnp.maximum(p[:, 3:4], 1.0)
    blk = p.shape[0]
    cout_ref[...] = jnp.concatenate(
        [(ssum / cnt) * COORDS_WEIGHT, jnp.zeros((blk, 13), jnp.float32)],
        axis=1)


def _node_mlp(h, pf0, pf1, pd0, pd1, wnh, wna, bn1, wn2, bn2, blk):
    n, d = h.shape
    wspec = pl.BlockSpec((128, 128), lambda i: (0, 0))
    bspec = pl.BlockSpec((1, 128), lambda i: (0, 0))
    nspec = pl.BlockSpec((blk, 128), lambda i: (i, 0))
    return pl.pallas_call(
        _k4_body,
        grid=(n // blk,),
        in_specs=[nspec, nspec, nspec,
                  pl.BlockSpec((blk, 16), lambda i: (i, 0)),
                  pl.BlockSpec((blk, 16), lambda i: (i, 0)),
                  wspec, wspec, bspec, wspec, bspec],
        out_specs=[
            pl.BlockSpec((blk, d), lambda i: (i, 0)),
            pl.BlockSpec((blk, 16), lambda i: (i, 0)),
        ],
        out_shape=[
            jax.ShapeDtypeStruct((n, d), jnp.float32),
            jax.ShapeDtypeStruct((n, 16), jnp.float32),
        ],
    )(h, pf0, pf1, pd0, pd1, wnh, wna, bn1, wn2, bn2)


def kernel(h, edge_index, coord, W_e1, b_e1, W_e2, b_e2, W_n1, b_n1, W_n2,
           b_n2, W_c1, b_c1, W_cl):
    n, d = h.shape
    e = edge_index.shape[1]
    ep = ((e + 1024 * NW - 1) // (1024 * NW)) * (1024 * NW)
    npad = 10240 if n <= 10240 else ((n + 2 * NS * 128 - 1)
                                     // (NS * 128)) * (NS * 128)
    dump = n

    row = edge_index[0]
    col = edge_index[1]
    pad = ep - e
    row_p = jnp.concatenate([row, jnp.zeros((pad,), jnp.int32)])
    col_p = jnp.concatenate([col, jnp.zeros((pad,), jnp.int32)])
    rowscat = jnp.concatenate([row, jnp.full((pad,), dump, jnp.int32)])

    cpad = jnp.concatenate([coord, jnp.zeros((n, 13), jnp.float32)], axis=1)

    w1a = W_e1[0:d]
    w1b = W_e1[d:2 * d]
    wr01 = (W_e1[2 * d] + W_e1[2 * d + 1])[None, :]
    wr2 = W_e1[2 * d + 2][None, :]

    ta, tb = _prep_tables(h, cpad, w1a, w1b, blk=2000)
    sa, sb = _gather_sum(ta, tb, row_p, col_p, ep, be=128,
                         frac0_num=1, frac0_den=2)
    t, td = _edge_mlp(sa, sb, wr01, wr2, b_e1[None, :], W_e2, b_e2[None, :],
                      W_c1, b_c1[None, :], W_cl.reshape(1, -1), blk=2048)
    pf, pd = _segment_sum(t, td, rowscat, npad, ep, be=64)
    h_out, c_out = _node_mlp(h, pf[0, :n], pf[1, :n], pd[0, :n], pd[1, :n],
                             W_n1[:d], W_n1[d:], b_n1[None, :], W_n2,
                             b_n2[None, :], blk=2000)
    return h_out, c_out[:, :3]


# speedup vs baseline: 1.3246x; 1.3246x over previous
import functools

import jax
import jax.numpy as jnp
from jax import lax
from jax.experimental import pallas as pl
from jax.experimental.pallas import tpu as pltpu
from jax.experimental.pallas import tpu_sc as plsc

EPS = 1e-8
COORDS_WEIGHT = 1.0

NC = 2
NS = 16
NW = NC * NS


def _pack_bf16(mm):
    lo = lax.bitcast_convert_type(mm[:, 0:64].astype(jnp.bfloat16),
                                  jnp.uint16).astype(jnp.uint32)
    hi = lax.bitcast_convert_type(mm[:, 64:128].astype(jnp.bfloat16),
                                  jnp.uint16).astype(jnp.uint32)
    return lax.bitcast_convert_type(lo | (hi << 16), jnp.int32)


def _k0_body(h_ref, cpad_ref, w1a_ref, w1b_ref, ta_ref, tb_ref):
    h = h_ref[...]
    cbits = lax.bitcast_convert_type(cpad_ref[...], jnp.int32)
    mma = jnp.dot(h, w1a_ref[...], preferred_element_type=jnp.float32)
    mmb = jnp.dot(h, w1b_ref[...], preferred_element_type=jnp.float32)
    ta_ref[...] = jnp.concatenate([_pack_bf16(mma), cbits], axis=1)
    tb_ref[...] = jnp.concatenate([_pack_bf16(mmb), cbits], axis=1)


def _prep_tables(h, cpad, w1a, w1b, blk):
    n, d = h.shape
    return pl.pallas_call(
        _k0_body,
        grid=(n // blk,),
        in_specs=[
            pl.BlockSpec((blk, d), lambda i: (i, 0)),
            pl.BlockSpec((blk, 16), lambda i: (i, 0)),
            pl.BlockSpec((d, d), lambda i: (0, 0)),
            pl.BlockSpec((d, d), lambda i: (0, 0)),
        ],
        out_specs=[
            pl.BlockSpec((blk, 80), lambda i: (i, 0)),
            pl.BlockSpec((blk, 80), lambda i: (i, 0)),
        ],
        out_shape=[
            jax.ShapeDtypeStruct((n, 80), jnp.int32),
            jax.ShapeDtypeStruct((n, 80), jnp.int32),
        ],
    )(h, cpad, w1a, w1b)


def _gather_sum(ta, tb, row_p, col_p, ep, be):
    ew = ep // NW
    nb = ew // be
    mesh = plsc.VectorSubcoreMesh(core_axis_name="c", subcore_axis_name="s")
    msk = jnp.int32(-65536)
    half = jnp.int32(0x8000)

    @functools.partial(
        pl.kernel,
        out_type=jax.ShapeDtypeStruct((ep, 128), jnp.int32),
        mesh=mesh,
        scratch_types=[
            pltpu.VMEM((be,), jnp.int32), pltpu.VMEM((be,), jnp.int32),
            pltpu.VMEM((be,), jnp.int32), pltpu.VMEM((be,), jnp.int32),
            pltpu.VMEM((be, 80), jnp.int32), pltpu.VMEM((be, 80), jnp.int32),
            pltpu.VMEM((be, 80), jnp.int32), pltpu.VMEM((be, 80), jnp.int32),
            pltpu.VMEM((be, 128), jnp.int32), pltpu.VMEM((be, 128), jnp.int32),
            pltpu.SemaphoreType.DMA, pltpu.SemaphoreType.DMA,
        ],
        compiler_params=pltpu.CompilerParams(use_tc_tiling_on_sc=False,
                                             needs_layout_passes=False),
    )
    def k1(ta_hbm, tb_hbm, row_hbm, col_hbm, s_hbm,
           idx_r0, idx_c0, idx_r1, idx_c1, bufa0, bufb0, bufa1, bufb1,
           bufs0, bufs1, sem0, sem1):
        c = lax.axis_index("c")
        s = lax.axis_index("s")
        wid = c * NS + s
        base = wid * ew
        idx = ((idx_r0, idx_c0), (idx_r1, idx_c1))
        buf = ((bufa0, bufb0), (bufa1, bufb1))
        out = (bufs0, bufs1)
        sem = (sem0, sem1)

        def load(k, off):
            pltpu.sync_copy(row_hbm.at[pl.ds(off, be)], idx[k][0])
            pltpu.sync_copy(col_hbm.at[pl.ds(off, be)], idx[k][1])
            pltpu.async_copy(ta_hbm.at[idx[k][0]], buf[k][0], sem[k])
            pltpu.async_copy(tb_hbm.at[idx[k][1]], buf[k][1], sem[k])

        def drain(k):
            pltpu.make_async_copy(ta_hbm.at[idx[k][0]], buf[k][0], sem[k]).wait()
            pltpu.make_async_copy(tb_hbm.at[idx[k][1]], buf[k][1], sem[k]).wait()

        def combine(k):
            bufa, bufb = buf[k]
            bufs = out[k]

            def rowfn(i, c2):
                for g in range(4):
                    sl = pl.ds(g * 16, 16)
                    a = bufa[i, sl]
                    b = bufb[i, sl]
                    slo = (plsc.bitcast(a << 16, jnp.float32)
                           + plsc.bitcast(b << 16, jnp.float32))
                    shi = (plsc.bitcast(a & msk, jnp.float32)
                           + plsc.bitcast(b & msk, jnp.float32))
                    lo16 = lax.shift_right_logical(
                        plsc.bitcast(slo, jnp.int32) + half, 16)
                    hi16 = (plsc.bitcast(shi, jnp.int32) + half) & msk
                    bufs[i, sl] = lo16 | hi16
                slc = pl.ds(64, 16)
                da = plsc.bitcast(bufa[i, slc], jnp.float32)
                db = plsc.bitcast(bufb[i, slc], jnp.float32)
                bufs[i, slc] = plsc.bitcast(da - db, jnp.int32)
                return c2

            lax.fori_loop(0, be, rowfn, 0)

        def store(k, off):
            pltpu.sync_copy(out[k], s_hbm.at[pl.ds(off, be)])

        load(0, base)

        def pair(i, carry):
            off0 = base + (2 * i) * be
            load(1, off0 + be)
            drain(0)
            combine(0)
            store(0, off0)

            @pl.when(i + 1 < nb // 2)
            def _():
                load(0, off0 + 2 * be)

            drain(1)
            combine(1)
            store(1, off0 + be)
            return carry

        lax.fori_loop(0, nb // 2, pair, 0)

    return k1(ta, tb, row_p, col_p)


def _unpack_bf16(p):
    lo = lax.bitcast_convert_type(p << 16, jnp.float32)
    hi = lax.bitcast_convert_type(p & jnp.int32(-65536), jnp.float32)
    return lo, hi


def _k2_body(s_ref, wr01_ref, wr2_ref, be1_ref, we2_ref, be2_ref,
             wc1_ref, bc1_ref, wclt_ref, t_ref, td_ref):
    x = s_ref[...]
    slo, shi = _unpack_bf16(x[:, 0:64])
    s = jnp.concatenate([slo, shi], axis=1)
    d = lax.bitcast_convert_type(x[:, 64:67], jnp.float32)
    sumsq = jnp.sum(d * d, axis=1, keepdims=True)
    nrm = jnp.sqrt(sumsq)
    fxv = sumsq / ((nrm + EPS) * (nrm + EPS))
    e1 = jax.nn.relu(s + nrm * wr01_ref[...] + fxv * wr2_ref[...] + be1_ref[...])
    ef = jax.nn.relu(
        jnp.dot(e1, we2_ref[...], preferred_element_type=jnp.float32)
        + be2_ref[...])
    c1 = jax.nn.relu(
        jnp.dot(ef, wc1_ref[...], preferred_element_type=jnp.float32)
        + bc1_ref[...])
    scalar = jnp.sum(c1 * wclt_ref[...], axis=1, keepdims=True)
    trans = jnp.clip(d * scalar, -100.0, 100.0)
    blk = s.shape[0]
    t_ref[...] = ef
    td_ref[...] = jnp.concatenate(
        [trans, jnp.ones((blk, 1), jnp.float32),
         jnp.zeros((blk, 124), jnp.float32)], axis=1)


def _edge_mlp(s, wr01, wr2, be1, we2, be2, wc1, bc1, wclt, blk):
    ep = s.shape[0]
    wspec = pl.BlockSpec((128, 128), lambda i: (0, 0))
    bspec = pl.BlockSpec((1, 128), lambda i: (0, 0))
    espec = pl.BlockSpec((blk, 128), lambda i: (i, 0))
    return pl.pallas_call(
        _k2_body,
        grid=(ep // blk,),
        in_specs=[espec,
                  bspec, bspec, bspec, wspec, bspec, wspec, bspec, bspec],
        out_specs=[espec, espec],
        out_shape=[
            jax.ShapeDtypeStruct((ep, 128), jnp.float32),
            jax.ShapeDtypeStruct((ep, 128), jnp.float32),
        ],
    )(s, wr01, wr2, be1, we2, be2, wc1, bc1, wclt)


def _segment_sum(t, td, rowscat, npad, ep, be):
    ew = ep // NW
    nb = ew // be
    rs = npad // NS
    ch = 64
    nch = rs // ch
    mesh = plsc.VectorSubcoreMesh(core_axis_name="c", subcore_axis_name="s")

    @functools.partial(
        pl.kernel,
        out_type=[
            jax.ShapeDtypeStruct((NC, npad, 128), jnp.float32),
            jax.ShapeDtypeStruct((NC, npad, 16), jnp.float32),
        ],
        mesh=mesh,
        scratch_types=[
            pltpu.VMEM((be,), jnp.int32), pltpu.VMEM((be,), jnp.int32),
            pltpu.VMEM((be, 128), jnp.float32),
            pltpu.VMEM((be, 128), jnp.float32),
            pltpu.VMEM((be, 16), jnp.float32),
            pltpu.VMEM((be, 16), jnp.float32),
            pltpu.VMEM((ch, 128), jnp.float32),
            pltpu.VMEM((ch, 16), jnp.float32),
            pltpu.VMEM_SHARED((npad, 128), jnp.float32),
            pltpu.VMEM_SHARED((npad, 16), jnp.float32),
            pltpu.SemaphoreType.DMA, pltpu.SemaphoreType.DMA,
        ],
        compiler_params=pltpu.CompilerParams(use_tc_tiling_on_sc=False),
    )
    def k3(t_hbm, td_hbm, row_hbm, outf_hbm, outd_hbm, idx_v0, idx_v1,
           buft0, buft1, bufd0, bufd1, bounce, bounced, acc, accd,
           sem0, sem1):
        c = lax.axis_index("c")
        s = lax.axis_index("s")
        base = c * (ep // NC) + s * ew
        zeros16 = jnp.zeros((16,), jnp.float32)
        idx = (idx_v0, idx_v1)
        buf = (buft0, buft1)
        bufd = (bufd0, bufd1)
        sem = (sem0, sem1)

        def zrow(i, c2):
            for j in range(8):
                bounce[i, pl.ds(j * 16, 16)] = zeros16
            bounced[i, pl.ds(0, 16)] = zeros16
            return c2

        def load(k, off):
            pltpu.async_copy(row_hbm.at[pl.ds(off, be)], idx[k], sem[k])
            pltpu.async_copy(t_hbm.at[pl.ds(off, be)], buf[k], sem[k])
            pltpu.async_copy(td_hbm.at[pl.ds(off, be), pl.ds(0, 16)],
                             bufd[k], sem[k])

        def drain(k, off):
            pltpu.make_async_copy(row_hbm.at[pl.ds(off, be)], idx[k],
                                  sem[k]).wait()
            pltpu.make_async_copy(t_hbm.at[pl.ds(off, be)], buf[k],
                                  sem[k]).wait()
            pltpu.make_async_copy(td_hbm.at[pl.ds(off, be), pl.ds(0, 16)],
                                  bufd[k], sem[k]).wait()

        lax.fori_loop(0, ch, zrow, 0)

        def zch(k, c2):
            pltpu.sync_copy(bounce, acc.at[pl.ds(s * rs + k * ch, ch)])
            pltpu.sync_copy(bounced, accd.at[pl.ds(s * rs + k * ch, ch)])
            return c2

        lax.fori_loop(0, nch, zch, 0)
        plsc.subcore_barrier()
        load(0, base)

        def pair(i, c2):
            off0 = base + (2 * i) * be
            load(1, off0 + be)
            drain(0, off0)
            pltpu.sync_copy(buf[0], acc.at[idx[0]], add=True)
            pltpu.sync_copy(bufd[0], accd.at[idx[0]], add=True)

            @pl.when(i + 1 < nb // 2)
            def _():
                load(0, off0 + 2 * be)

            drain(1, off0 + be)
            pltpu.sync_copy(buf[1], acc.at[idx[1]], add=True)
            pltpu.sync_copy(bufd[1], accd.at[idx[1]], add=True)
            return c2

        lax.fori_loop(0, nb // 2, pair, 0)
        plsc.subcore_barrier()

        def och(k, c2):
            r0 = s * rs + k * ch
            pltpu.sync_copy(acc.at[pl.ds(r0, ch)], bounce)
            pltpu.sync_copy(bounce, outf_hbm.at[c, pl.ds(r0, ch)])
            pltpu.sync_copy(accd.at[pl.ds(r0, ch)], bounced)
            pltpu.sync_copy(bounced, outd_hbm.at[c, pl.ds(r0, ch)])
            return c2

        lax.fori_loop(0, nch, och, 0)

    return k3(t, td, rowscat)


def _k4_body(h_ref, pf0_ref, pf1_ref, pd0_ref, pd1_ref, wnh_ref, wna_ref,
             bn1_ref, wn2_ref, bn2_ref, hout_ref, cout_ref):
    agg = pf0_ref[...] + pf1_ref[...]
    h = h_ref[...]
    n1 = jax.nn.relu(
        jnp.dot(h, wnh_ref[...], preferred_element_type=jnp.float32)
        + jnp.dot(agg, wna_ref[...], preferred_element_type=jnp.float32)
        + bn1_ref[...])
    hout_ref[...] = (
        jnp.dot(n1, wn2_ref[...], preferred_element_type=jnp.float32)
        + bn2_ref[...])
    p = pd0_ref[...] + pd1_ref[...]
    ssum = p[:, 0:3]
    cnt = jnp.maximum(p[:, 3:4], 1.0)
    blk = p.shape[0]
    cout_ref[...] = jnp.concatenate(
        [(ssum / cnt) * COORDS_WEIGHT, jnp.zeros((blk, 13), jnp.float32)],
        axis=1)


def _node_mlp(h, pf0, pf1, pd0, pd1, wnh, wna, bn1, wn2, bn2, blk):
    n, d = h.shape
    wspec = pl.BlockSpec((128, 128), lambda i: (0, 0))
    bspec = pl.BlockSpec((1, 128), lambda i: (0, 0))
    nspec = pl.BlockSpec((blk, 128), lambda i: (i, 0))
    return pl.pallas_call(
        _k4_body,
        grid=(n // blk,),
        in_specs=[nspec, nspec, nspec,
                  pl.BlockSpec((blk, 16), lambda i: (i, 0)),
                  pl.BlockSpec((blk, 16), lambda i: (i, 0)),
                  wspec, wspec, bspec, wspec, bspec],
        out_specs=[
            pl.BlockSpec((blk, d), lambda i: (i, 0)),
            pl.BlockSpec((blk, 16), lambda i: (i, 0)),
        ],
        out_shape=[
            jax.ShapeDtypeStruct((n, d), jnp.float32),
            jax.ShapeDtypeStruct((n, 16), jnp.float32),
        ],
    )(h, pf0, pf1, pd0, pd1, wnh, wna, bn1, wn2, bn2)


def kernel(h, edge_index, coord, W_e1, b_e1, W_e2, b_e2, W_n1, b_n1, W_n2,
           b_n2, W_c1, b_c1, W_cl):
    n, d = h.shape
    e = edge_index.shape[1]
    ep = ((e + 1024 * NW - 1) // (1024 * NW)) * (1024 * NW)
    npad = 10240 if n <= 10240 else ((n + 2 * NS * 128 - 1)
                                     // (NS * 128)) * (NS * 128)
    dump = n

    row = edge_index[0]
    col = edge_index[1]
    pad = ep - e
    row_p = jnp.concatenate([row, jnp.zeros((pad,), jnp.int32)])
    col_p = jnp.concatenate([col, jnp.zeros((pad,), jnp.int32)])
    rowscat = jnp.concatenate([row, jnp.full((pad,), dump, jnp.int32)])

    cpad = jnp.concatenate([coord, jnp.zeros((n, 13), jnp.float32)], axis=1)

    w1a = W_e1[0:d]
    w1b = W_e1[d:2 * d]
    wr01 = (W_e1[2 * d] + W_e1[2 * d + 1])[None, :]
    wr2 = W_e1[2 * d + 2][None, :]

    ta, tb = _prep_tables(h, cpad, w1a, w1b, blk=2000)
    s = _gather_sum(ta, tb, row_p, col_p, ep, be=128)
    t, td = _edge_mlp(s, wr01, wr2, b_e1[None, :], W_e2, b_e2[None, :],
                      W_c1, b_c1[None, :], W_cl.reshape(1, -1), blk=2048)
    pf, pd = _segment_sum(t, td, rowscat, npad, ep, be=64)
    h_out, c_out = _node_mlp(h, pf[0, :n], pf[1, :n], pd[0, :n], pd[1, :n],
                             W_n1[:d], W_n1[d:], b_n1[None, :], W_n2,
                             b_n2[None, :], blk=2000)
    return h_out, c_out[:, :3]
